# Initial kernel scaffold; baseline (speedup 1.0000x reference)
import jax
import jax.numpy as jnp
from jax.experimental import pallas as pl


def kernel(x, edge_index, edge_type, W1, b1, W2, b2, W3, b3, Wl, bl, W0, b0, Wx, bx):
    raise NotImplementedError("write your pallas kernel here")


# speedup vs baseline: 3.9184x; 3.9184x over previous
import functools

import jax
import jax.numpy as jnp
from jax import lax
from jax.experimental import pallas as pl
from jax.experimental.pallas import tpu as pltpu
from jax.experimental.pallas import tpu_sc as plsc

N = 10000
E = 320000
D = 128
HID = 64

NC = 2
NS = 16
NW = NC * NS
CH = 128
K = -(-E // (NW * CH * 8)) * 8
E_PAD = NW * K * CH
AGG_ROWS = 10240
ROWS_PER_TILE = AGG_ROWS // NS
G_PAD_ROWS = 8


def _dense_a_body(x_ref, w1, b1, w2, b2, w3, b3, wl, w0, wx, bd,
                  g_ref, d_ref):
    x = x_ref[...]
    hp = functools.partial(jnp.dot, preferred_element_type=jnp.float32,
                           precision=lax.Precision.HIGHEST)
    h = jnp.maximum(hp(x, w1[...]) + b1[...], 0.0)
    h = jnp.maximum(hp(h, w2[...]) + b2[...], 0.0)
    h = hp(h, w3[...]) + b3[...]
    g_ref[...] = hp(h, wl[...])
    d_ref[...] = hp(h, w0[...]) + hp(x, wx[...]) + bd[...]


def _final_body(p_ref, d_ref, o_ref):
    o_ref[...] = jnp.maximum(p_ref[0] + p_ref[1] + d_ref[...], 0.0)


def _sc_scatter_body(g_hbm, col_hbm, row_hbm, out_hbm,
                     col_buf, row_buf, rows_v, agg_sh, sem):
    cid = lax.axis_index("c")
    sid = lax.axis_index("s")
    wid = cid * NS + sid

    zero16 = jnp.zeros((16,), jnp.float32)

    def zbody(i, c):
        for j in range(HID // 16):
            rows_v[i, pl.ds(j * 16, 16)] = zero16
        return c

    lax.fori_loop(0, CH, zbody, 0)
    base = sid * ROWS_PER_TILE
    for t in range(ROWS_PER_TILE // CH):
        pltpu.sync_copy(rows_v, agg_sh.at[pl.ds(base + t * CH, CH)])

    pltpu.sync_copy(col_hbm.at[pl.ds(wid * K, K)], col_buf)
    pltpu.sync_copy(row_hbm.at[pl.ds(wid * K, K)], row_buf)
    plsc.subcore_barrier()

    def ebody(i, c):
        pltpu.async_copy(g_hbm.at[col_buf.at[i]], rows_v, sem).wait()
        pltpu.sync_copy(rows_v, agg_sh.at[row_buf.at[i]], add=True)
        return c

    lax.fori_loop(0, K, ebody, 0)
    plsc.subcore_barrier()

    pltpu.sync_copy(agg_sh.at[pl.ds(base, ROWS_PER_TILE)],
                    out_hbm.at[cid, pl.ds(base, ROWS_PER_TILE)])


_sc_scatter = pl.kernel(
    _sc_scatter_body,
    out_type=jax.ShapeDtypeStruct((NC, AGG_ROWS, HID), jnp.float32),
    mesh=plsc.VectorSubcoreMesh(core_axis_name="c", subcore_axis_name="s"),
    scratch_types=[
        pltpu.VMEM((K, CH), jnp.int32),
        pltpu.VMEM((K, CH), jnp.int32),
        pltpu.VMEM((CH, HID), jnp.float32),
        pltpu.VMEM_SHARED((AGG_ROWS, HID), jnp.float32),
        pltpu.SemaphoreType.DMA,
    ],
    compiler_params=pltpu.CompilerParams(use_tc_tiling_on_sc=False),
)


def kernel(x, edge_index, edge_type, W1, b1, W2, b2, W3, b3,
           Wl, bl, W0, b0, Wx, bx):
    BR = 1000
    grid = (N // BR,)
    full = lambda shape: pl.BlockSpec(shape, lambda i: (0,) * len(shape))
    bd = (bl + b0 + bx).reshape(1, HID)

    g, d = pl.pallas_call(
        _dense_a_body,
        grid=grid,
        in_specs=[
            pl.BlockSpec((BR, D), lambda i: (i, 0)),
            full((D, HID)), full((1, HID)),
            full((HID, HID)), full((1, HID)),
            full((HID, D)), full((1, D)),
            full((D, HID)), full((D, HID)), full((D, HID)),
            full((1, HID)),
        ],
        out_specs=[
            pl.BlockSpec((BR, HID), lambda i: (i, 0)),
            pl.BlockSpec((BR, HID), lambda i: (i, 0)),
        ],
        out_shape=[
            jax.ShapeDtypeStruct((N, HID), jnp.float32),
            jax.ShapeDtypeStruct((N, HID), jnp.float32),
        ],
    )(x, W1, b1.reshape(1, HID), W2, b2.reshape(1, HID),
      W3, b3.reshape(1, D), Wl, W0, Wx, bd)

    ei = edge_index[1]
    row, col = ei[0], ei[1]
    pad = E_PAD - E
    colp = jnp.concatenate([col, jnp.full((pad,), N, jnp.int32)])
    rowp = jnp.concatenate([row, jnp.zeros((pad,), jnp.int32)])
    g_ext = jnp.concatenate([g, jnp.zeros((G_PAD_ROWS, HID), jnp.float32)])

    partials = _sc_scatter(g_ext,
                           colp.reshape(NW * K, CH),
                           rowp.reshape(NW * K, CH))

    out = pl.pallas_call(
        _final_body,
        grid=grid,
        in_specs=[
            pl.BlockSpec((NC, BR, HID), lambda i: (0, i, 0)),
            pl.BlockSpec((BR, HID), lambda i: (i, 0)),
        ],
        out_specs=pl.BlockSpec((BR, HID), lambda i: (i, 0)),
        out_shape=jax.ShapeDtypeStruct((N, HID), jnp.float32),
    )(partials, d)
    return out


# speedup vs baseline: 4.5963x; 1.1730x over previous
import functools

import jax
import jax.numpy as jnp
from jax import lax
from jax.experimental import pallas as pl
from jax.experimental.pallas import tpu as pltpu
from jax.experimental.pallas import tpu_sc as plsc

N = 10000
E = 320000
D = 128
HID = 64

NC = 2
NS = 16
NW = NC * NS
CH = 128
K = -(-E // (NW * CH * 8)) * 8
E_PAD = NW * K * CH
AGG_ROWS = 10240
ROWS_PER_TILE = AGG_ROWS // NS
NBUF = 8
PD = 4


def _dense_a_body(x_ref, w1, b1, w2, b2, w3, b3, wl, w0, wx, bd,
                  g_ref, d_ref):
    x = x_ref[...]
    hp = functools.partial(jnp.dot, preferred_element_type=jnp.float32,
                           precision=lax.Precision.HIGHEST)
    h = jnp.maximum(hp(x, w1[...]) + b1[...], 0.0)
    h = jnp.maximum(hp(h, w2[...]) + b2[...], 0.0)
    h = hp(h, w3[...]) + b3[...]
    g_ref[...] = hp(h, wl[...])
    d_ref[...] = hp(h, w0[...]) + hp(x, wx[...]) + bd[...]


def _final_body(p_ref, d_ref, o_ref):
    o_ref[...] = jnp.maximum(p_ref[0] + p_ref[1] + d_ref[...], 0.0)


def _sc_scatter_body(g_hbm, col_hbm, row_hbm, out_hbm,
                     col_buf, row_buf, rows_v, agg_sh, gsem, ssem):
    cid = lax.axis_index("c")
    sid = lax.axis_index("s")
    wid = cid * NS + sid

    zero16 = jnp.zeros((16,), jnp.float32)

    def zbody(i, c):
        for j in range(HID // 16):
            rows_v[0, i, pl.ds(j * 16, 16)] = zero16
        return c

    lax.fori_loop(0, CH, zbody, 0)
    base = sid * ROWS_PER_TILE
    for t in range(ROWS_PER_TILE // CH):
        pltpu.sync_copy(rows_v.at[0], agg_sh.at[pl.ds(base + t * CH, CH)])

    pltpu.sync_copy(col_hbm.at[pl.ds(wid * K, K)], col_buf)
    pltpu.sync_copy(row_hbm.at[pl.ds(wid * K, K)], row_buf)
    plsc.subcore_barrier()

    def gather(j, bj):
        return pltpu.async_copy(g_hbm.at[col_buf.at[j]], rows_v.at[bj],
                                gsem.at[bj])

    def scatter(i, b):
        return pltpu.async_copy(rows_v.at[b], agg_sh.at[row_buf.at[i]],
                                ssem.at[b], add=True)

    for b in range(PD):
        gather(b, b)

    def round_body(g, c):
        for b in range(NBUF):
            i = g * NBUF + b
            pltpu.make_async_copy(g_hbm.at[col_buf.at[i]], rows_v.at[b],
                                  gsem.at[b]).wait()
            scatter(i, b)
            j = i + PD
            bj = (b + PD) % NBUF

            @pl.when(jnp.logical_and(j >= NBUF, j < K))
            def _():
                pltpu.make_async_copy(rows_v.at[bj],
                                      agg_sh.at[row_buf.at[i]],
                                      ssem.at[bj]).wait()

            @pl.when(j < K)
            def _():
                gather(j, bj)
        return c

    lax.fori_loop(0, K // NBUF, round_body, 0)
    for b in range(NBUF):
        pltpu.make_async_copy(rows_v.at[b], agg_sh.at[row_buf.at[0]],
                              ssem.at[b]).wait()
    plsc.subcore_barrier()

    pltpu.sync_copy(agg_sh.at[pl.ds(base, ROWS_PER_TILE)],
                    out_hbm.at[cid, pl.ds(base, ROWS_PER_TILE)])


_sc_scatter = pl.kernel(
    _sc_scatter_body,
    out_type=jax.ShapeDtypeStruct((NC, AGG_ROWS, HID), jnp.float32),
    mesh=plsc.VectorSubcoreMesh(core_axis_name="c", subcore_axis_name="s"),
    scratch_types=[
        pltpu.VMEM((K, CH), jnp.int32),
        pltpu.VMEM((K, CH), jnp.int32),
        pltpu.VMEM((NBUF, CH, HID), jnp.float32),
        pltpu.VMEM_SHARED((AGG_ROWS, HID), jnp.float32),
        pltpu.SemaphoreType.DMA((NBUF,)),
        pltpu.SemaphoreType.DMA((NBUF,)),
    ],
    compiler_params=pltpu.CompilerParams(use_tc_tiling_on_sc=False),
)


def kernel(x, edge_index, edge_type, W1, b1, W2, b2, W3, b3,
           Wl, bl, W0, b0, Wx, bx):
    BR = 1000
    grid = (N // BR,)
    full = lambda shape: pl.BlockSpec(shape, lambda i: (0,) * len(shape))
    bd = (bl + b0 + bx).reshape(1, HID)

    g, d = pl.pallas_call(
        _dense_a_body,
        grid=grid,
        in_specs=[
            pl.BlockSpec((BR, D), lambda i: (i, 0)),
            full((D, HID)), full((1, HID)),
            full((HID, HID)), full((1, HID)),
            full((HID, D)), full((1, D)),
            full((D, HID)), full((D, HID)), full((D, HID)),
            full((1, HID)),
        ],
        out_specs=[
            pl.BlockSpec((BR, HID), lambda i: (i, 0)),
            pl.BlockSpec((BR, HID), lambda i: (i, 0)),
        ],
        out_shape=[
            jax.ShapeDtypeStruct((N, HID), jnp.float32),
            jax.ShapeDtypeStruct((N, HID), jnp.float32),
        ],
    )(x, W1, b1.reshape(1, HID), W2, b2.reshape(1, HID),
      W3, b3.reshape(1, D), Wl, W0, Wx, bd)

    ei = edge_index[1]
    row, col = ei[0], ei[1]
    pad = E_PAD - E
    colp = jnp.concatenate([col, jnp.zeros((pad,), jnp.int32)])
    rowp = jnp.concatenate([row, jnp.full((pad,), AGG_ROWS - 1, jnp.int32)])

    partials = _sc_scatter(g,
                           colp.reshape(NW * K, CH),
                           rowp.reshape(NW * K, CH))

    out = pl.pallas_call(
        _final_body,
        grid=grid,
        in_specs=[
            pl.BlockSpec((NC, BR, HID), lambda i: (0, i, 0)),
            pl.BlockSpec((BR, HID), lambda i: (i, 0)),
        ],
        out_specs=pl.BlockSpec((BR, HID), lambda i: (i, 0)),
        out_shape=jax.ShapeDtypeStruct((N, HID), jnp.float32),
    )(partials, d)
    return out


# speedup vs baseline: 11.7517x; 2.5568x over previous
import functools

import jax
import jax.numpy as jnp
from jax import lax
from jax.experimental import pallas as pl
from jax.experimental.pallas import tpu as pltpu
from jax.experimental.pallas import tpu_sc as plsc

N = 10000
E = 320000
D = 128
HID = 64

NC = 2
NS = 16
NW = NC * NS
CH = 128
K = -(-E // (NW * CH * 8)) * 8
E_PAD = NW * K * CH
AGG_ROWS = 10240
ROWS_PER_TILE = AGG_ROWS // NS
NBUF = 8
PD = 4


def _dense_a_body(x_ref, w1, b1, w2, b2, w3, b3, wl, w0, wx, bd,
                  g_ref, d_ref):
    x = x_ref[...]
    hp = functools.partial(jnp.dot, preferred_element_type=jnp.float32)
    h = jnp.maximum(hp(x, w1[...]) + b1[...], 0.0)
    h = jnp.maximum(hp(h, w2[...]) + b2[...], 0.0)
    h = hp(h, w3[...]) + b3[...]
    g_ref[...] = hp(h, wl[...])
    d_ref[...] = hp(h, w0[...]) + hp(x, wx[...]) + bd[...]


def _final_body(p_ref, d_ref, o_ref):
    o_ref[...] = jnp.maximum(p_ref[0] + p_ref[1] + d_ref[...], 0.0)


def _sc_scatter_body(g_hbm, col_hbm, row_hbm, out_hbm,
                     col_buf, row_buf, rows_v, agg_sh, gsem, ssem):
    cid = lax.axis_index("c")
    sid = lax.axis_index("s")
    wid = cid * NS + sid

    zero16 = jnp.zeros((16,), jnp.float32)

    def zbody(i, c):
        for j in range(HID // 16):
            rows_v[0, i, pl.ds(j * 16, 16)] = zero16
        return c

    lax.fori_loop(0, CH, zbody, 0)
    base = sid * ROWS_PER_TILE
    for t in range(ROWS_PER_TILE // CH):
        pltpu.sync_copy(rows_v.at[0], agg_sh.at[pl.ds(base + t * CH, CH)])

    pltpu.sync_copy(col_hbm.at[pl.ds(wid * K, K)], col_buf)
    pltpu.sync_copy(row_hbm.at[pl.ds(wid * K, K)], row_buf)
    plsc.subcore_barrier()

    def gather(j, bj):
        return pltpu.async_copy(g_hbm.at[col_buf.at[j]], rows_v.at[bj],
                                gsem.at[bj])

    def scatter(i, b):
        return pltpu.async_copy(rows_v.at[b], agg_sh.at[row_buf.at[i]],
                                ssem.at[b], add=True)

    for b in range(PD):
        gather(b, b)

    def round_body(g, c):
        for b in range(NBUF):
            i = g * NBUF + b
            pltpu.make_async_copy(g_hbm.at[col_buf.at[i]], rows_v.at[b],
                                  gsem.at[b]).wait()
            scatter(i, b)
            j = i + PD
            bj = (b + PD) % NBUF

            @pl.when(jnp.logical_and(j >= NBUF, j < K))
            def _():
                pltpu.make_async_copy(rows_v.at[bj],
                                      agg_sh.at[row_buf.at[i]],
                                      ssem.at[bj]).wait()

            @pl.when(j < K)
            def _():
                gather(j, bj)
        return c

    lax.fori_loop(0, K // NBUF, round_body, 0)
    for b in range(NBUF):
        pltpu.make_async_copy(rows_v.at[b], agg_sh.at[row_buf.at[0]],
                              ssem.at[b]).wait()
    plsc.subcore_barrier()

    pltpu.sync_copy(agg_sh.at[pl.ds(base, ROWS_PER_TILE)],
                    out_hbm.at[cid, pl.ds(base, ROWS_PER_TILE)])


_sc_scatter = pl.kernel(
    _sc_scatter_body,
    out_type=jax.ShapeDtypeStruct((NC, AGG_ROWS, HID), jnp.float32),
    mesh=plsc.VectorSubcoreMesh(core_axis_name="c", subcore_axis_name="s"),
    scratch_types=[
        pltpu.VMEM((K, CH), jnp.int32),
        pltpu.VMEM((K, CH), jnp.int32),
        pltpu.VMEM((NBUF, CH, HID), jnp.float32),
        pltpu.VMEM_SHARED((AGG_ROWS, HID), jnp.float32),
        pltpu.SemaphoreType.DMA((NBUF,)),
        pltpu.SemaphoreType.DMA((NBUF,)),
    ],
    compiler_params=pltpu.CompilerParams(use_tc_tiling_on_sc=False),
)


def kernel(x, edge_index, edge_type, W1, b1, W2, b2, W3, b3,
           Wl, bl, W0, b0, Wx, bx):
    BR = 1000
    grid = (N // BR,)
    full = lambda shape: pl.BlockSpec(shape, lambda i: (0,) * len(shape))
    bd = (bl + b0 + bx).reshape(1, HID)

    g, d = pl.pallas_call(
        _dense_a_body,
        grid=grid,
        in_specs=[
            pl.BlockSpec((BR, D), lambda i: (i, 0)),
            full((D, HID)), full((1, HID)),
            full((HID, HID)), full((1, HID)),
            full((HID, D)), full((1, D)),
            full((D, HID)), full((D, HID)), full((D, HID)),
            full((1, HID)),
        ],
        out_specs=[
            pl.BlockSpec((BR, HID), lambda i: (i, 0)),
            pl.BlockSpec((BR, HID), lambda i: (i, 0)),
        ],
        out_shape=[
            jax.ShapeDtypeStruct((N, HID), jnp.float32),
            jax.ShapeDtypeStruct((N, HID), jnp.float32),
        ],
    )(x, W1, b1.reshape(1, HID), W2, b2.reshape(1, HID),
      W3, b3.reshape(1, D), Wl, W0, Wx, bd)

    ei = edge_index[1]
    row, col = ei[0], ei[1]
    pad = E_PAD - E
    it = jnp.arange(pad, dtype=jnp.int32)
    colp = jnp.concatenate([col, it % N])
    rowp = jnp.concatenate([row, N + it % (AGG_ROWS - N)])

    partials = _sc_scatter(g,
                           colp.reshape(NW * K, CH),
                           rowp.reshape(NW * K, CH))

    out = pl.pallas_call(
        _final_body,
        grid=grid,
        in_specs=[
            pl.BlockSpec((NC, BR, HID), lambda i: (0, i, 0)),
            pl.BlockSpec((BR, HID), lambda i: (i, 0)),
        ],
        out_specs=pl.BlockSpec((BR, HID), lambda i: (i, 0)),
        out_shape=jax.ShapeDtypeStruct((N, HID), jnp.float32),
    )(partials, d)
    return out


# speedup vs baseline: 13.5136x; 1.1499x over previous
import functools

import jax
import jax.numpy as jnp
from jax import lax
from jax.experimental import pallas as pl
from jax.experimental.pallas import tpu as pltpu
from jax.experimental.pallas import tpu_sc as plsc

N = 10000
E = 320000
D = 128
HID = 64

NC = 2
NS = 16
NW = NC * NS
CH = 128
NCHUNK = E // CH
KBASE = NCHUNK // NW
KREM = NCHUNK - NW * KBASE
KMAX = KBASE + 1
AGG_ROWS = 10240
ROWS_PER_TILE = AGG_ROWS // NS
NBUF = 8
PD = 4
NROUND = -(-KMAX // NBUF)


def _dense_a_body(x_ref, w1, b1, w2, b2, w3, b3, wl, w0, wx, bd,
                  g_ref, d_ref):
    x = x_ref[...]
    hp = functools.partial(jnp.dot, preferred_element_type=jnp.float32)
    h = jnp.maximum(hp(x, w1[...]) + b1[...], 0.0)
    h = jnp.maximum(hp(h, w2[...]) + b2[...], 0.0)
    h = hp(h, w3[...]) + b3[...]
    g_ref[...] = hp(h, wl[...])
    d_ref[...] = hp(h, w0[...]) + hp(x, wx[...]) + bd[...]


def _final_body(p_ref, d_ref, o_ref):
    o_ref[...] = jnp.maximum(p_ref[0] + p_ref[1] + d_ref[...], 0.0)


def _sc_scatter_body(g_hbm, idx_hbm, out_hbm,
                     eb, rows_v, agg_sh, gsem, ssem):
    cid = lax.axis_index("c")
    sid = lax.axis_index("s")
    wid = cid * NS + sid
    kw = KBASE + jnp.where(wid >= NW - KREM, 1, 0)
    start = KBASE * wid + jnp.maximum(wid - (NW - KREM), 0)

    zero16 = jnp.zeros((16,), jnp.float32)

    def zbody(i, c):
        for j in range(HID // 16):
            rows_v[0, i, pl.ds(j * 16, 16)] = zero16
        return c

    lax.fori_loop(0, CH, zbody, 0)
    base = sid * ROWS_PER_TILE
    for t in range(ROWS_PER_TILE // CH):
        pltpu.sync_copy(rows_v.at[0], agg_sh.at[pl.ds(base + t * CH, CH)])

    pltpu.sync_copy(idx_hbm.at[1, pl.ds(start, KMAX)], eb.at[pl.ds(0, KMAX)])
    pltpu.sync_copy(idx_hbm.at[1, pl.ds(NCHUNK + start, KMAX)],
                    eb.at[pl.ds(KMAX, KMAX)])
    plsc.subcore_barrier()

    def gather(j, bj):
        return pltpu.async_copy(g_hbm.at[eb.at[KMAX + j]], rows_v.at[bj],
                                gsem.at[bj])

    def scatter(i, b):
        return pltpu.async_copy(rows_v.at[b], agg_sh.at[eb.at[i]],
                                ssem.at[b], add=True)

    for b in range(PD):
        gather(b, b)

    def round_body(g, c):
        for b in range(NBUF):
            i = g * NBUF + b
            j = i + PD
            bj = (b + PD) % NBUF

            @pl.when(i < kw)
            def _():
                pltpu.make_async_copy(g_hbm.at[eb.at[KMAX + i]],
                                      rows_v.at[b], gsem.at[b]).wait()
                scatter(i, b)

            @pl.when(jnp.logical_and(j >= NBUF, j < kw))
            def _():
                pltpu.make_async_copy(rows_v.at[bj], agg_sh.at[eb.at[0]],
                                      ssem.at[bj]).wait()

            @pl.when(j < kw)
            def _():
                gather(j, bj)
        return c

    lax.fori_loop(0, NROUND, round_body, 0)
    for b in range(NBUF):
        pltpu.make_async_copy(rows_v.at[b], agg_sh.at[eb.at[0]],
                              ssem.at[b]).wait()
    plsc.subcore_barrier()

    pltpu.sync_copy(agg_sh.at[pl.ds(base, ROWS_PER_TILE)],
                    out_hbm.at[cid, pl.ds(base, ROWS_PER_TILE)])


_sc_scatter = pl.kernel(
    _sc_scatter_body,
    out_type=jax.ShapeDtypeStruct((NC, AGG_ROWS, HID), jnp.float32),
    mesh=plsc.VectorSubcoreMesh(core_axis_name="c", subcore_axis_name="s"),
    scratch_types=[
        pltpu.VMEM((2 * KMAX, CH), jnp.int32),
        pltpu.VMEM((NBUF, CH, HID), jnp.float32),
        pltpu.VMEM_SHARED((AGG_ROWS, HID), jnp.float32),
        pltpu.SemaphoreType.DMA((NBUF,)),
        pltpu.SemaphoreType.DMA((NBUF,)),
    ],
    compiler_params=pltpu.CompilerParams(use_tc_tiling_on_sc=False),
)


def kernel(x, edge_index, edge_type, W1, b1, W2, b2, W3, b3,
           Wl, bl, W0, b0, Wx, bx):
    BR = 1000
    grid = (N // BR,)
    full = lambda shape: pl.BlockSpec(shape, lambda i: (0,) * len(shape))
    bd = (bl + b0 + bx).reshape(1, HID)

    g, d = pl.pallas_call(
        _dense_a_body,
        grid=grid,
        in_specs=[
            pl.BlockSpec((BR, D), lambda i: (i, 0)),
            full((D, HID)), full((1, HID)),
            full((HID, HID)), full((1, HID)),
            full((HID, D)), full((1, D)),
            full((D, HID)), full((D, HID)), full((D, HID)),
            full((1, HID)),
        ],
        out_specs=[
            pl.BlockSpec((BR, HID), lambda i: (i, 0)),
            pl.BlockSpec((BR, HID), lambda i: (i, 0)),
        ],
        out_shape=[
            jax.ShapeDtypeStruct((N, HID), jnp.float32),
            jax.ShapeDtypeStruct((N, HID), jnp.float32),
        ],
    )(x, W1, b1.reshape(1, HID), W2, b2.reshape(1, HID),
      W3, b3.reshape(1, D), Wl, W0, Wx, bd)

    partials = _sc_scatter(g, edge_index.reshape(2, 2 * NCHUNK, CH))

    out = pl.pallas_call(
        _final_body,
        grid=grid,
        in_specs=[
            pl.BlockSpec((NC, BR, HID), lambda i: (0, i, 0)),
            pl.BlockSpec((BR, HID), lambda i: (i, 0)),
        ],
        out_specs=pl.BlockSpec((BR, HID), lambda i: (i, 0)),
        out_shape=jax.ShapeDtypeStruct((N, HID), jnp.float32),
    )(partials, d)
    return out


# speedup vs baseline: 15.1102x; 1.1181x over previous
import functools

import jax
import jax.numpy as jnp
from jax import lax
from jax.experimental import pallas as pl
from jax.experimental.pallas import tpu as pltpu
from jax.experimental.pallas import tpu_sc as plsc

N = 10000
E = 320000
D = 128
HID = 64

NC = 2
NS = 16
NW = NC * NS
CH = 128
NCHUNK = E // CH
KBASE = NCHUNK // NW
KREM = NCHUNK - NW * KBASE
KMAX = KBASE + 1
AGG_ROWS = 10240
ROWS_PER_TILE = AGG_ROWS // NS
NBUF = 8
PD = 4
NROUND = -(-KMAX // NBUF)


def _dense_a_body(x_ref, w1t, b1, w2, b2, w3, b3, wlt, w0t, wxt, bd,
                  g_ref, d_ref):
    x = x_ref[...]
    hp = functools.partial(jnp.dot, preferred_element_type=jnp.float32)
    hpt = functools.partial(lax.dot_general,
                            dimension_numbers=(((1,), (1,)), ((), ())),
                            preferred_element_type=jnp.float32)
    h = jnp.maximum(hpt(x, w1t[...]) + b1[...], 0.0)
    h = jnp.maximum(hp(h, w2[...]) + b2[...], 0.0)
    h = hp(h, w3[...]) + b3[...]
    g_ref[...] = hpt(h, wlt[...])
    d_ref[...] = hpt(h, w0t[...]) + hpt(x, wxt[...]) + bd[...]


def _final_body(p_ref, d_ref, o_ref):
    o_ref[...] = jnp.maximum(p_ref[0] + p_ref[1] + d_ref[...], 0.0).T


def _sc_scatter_body(g_hbm, idx_hbm, out_hbm,
                     eb, rows_v, agg_sh, gsem, ssem):
    cid = lax.axis_index("c")
    sid = lax.axis_index("s")
    wid = cid * NS + sid
    kw = KBASE + jnp.where(wid >= NW - KREM, 1, 0)
    start = KBASE * wid + jnp.maximum(wid - (NW - KREM), 0)

    zero16 = jnp.zeros((16,), jnp.float32)

    def zbody(i, c):
        for j in range(HID // 16):
            rows_v[0, i, pl.ds(j * 16, 16)] = zero16
        return c

    lax.fori_loop(0, CH, zbody, 0)
    base = sid * ROWS_PER_TILE
    for t in range(ROWS_PER_TILE // CH):
        pltpu.sync_copy(rows_v.at[0], agg_sh.at[pl.ds(base + t * CH, CH)])

    pltpu.sync_copy(idx_hbm.at[1, pl.ds(start, KMAX)], eb.at[pl.ds(0, KMAX)])
    pltpu.sync_copy(idx_hbm.at[1, pl.ds(NCHUNK + start, KMAX)],
                    eb.at[pl.ds(KMAX, KMAX)])
    plsc.subcore_barrier()

    def gather(j, bj):
        return pltpu.async_copy(g_hbm.at[eb.at[KMAX + j]], rows_v.at[bj],
                                gsem.at[bj])

    def scatter(i, b):
        return pltpu.async_copy(rows_v.at[b], agg_sh.at[eb.at[i]],
                                ssem.at[b], add=True)

    for b in range(PD):
        gather(b, b)

    def round_body(g, c):
        for b in range(NBUF):
            i = g * NBUF + b
            j = i + PD
            bj = (b + PD) % NBUF

            @pl.when(i < kw)
            def _():
                pltpu.make_async_copy(g_hbm.at[eb.at[KMAX + i]],
                                      rows_v.at[b], gsem.at[b]).wait()
                scatter(i, b)

            @pl.when(jnp.logical_and(j >= NBUF, j < kw))
            def _():
                pltpu.make_async_copy(rows_v.at[bj], agg_sh.at[eb.at[0]],
                                      ssem.at[bj]).wait()

            @pl.when(j < kw)
            def _():
                gather(j, bj)
        return c

    lax.fori_loop(0, NROUND, round_body, 0)
    for b in range(NBUF):
        pltpu.make_async_copy(rows_v.at[b], agg_sh.at[eb.at[0]],
                              ssem.at[b]).wait()
    plsc.subcore_barrier()

    pltpu.sync_copy(agg_sh.at[pl.ds(base, ROWS_PER_TILE)],
                    out_hbm.at[cid, pl.ds(base, ROWS_PER_TILE)])


_sc_scatter = pl.kernel(
    _sc_scatter_body,
    out_type=jax.ShapeDtypeStruct((NC, AGG_ROWS, HID), jnp.float32),
    mesh=plsc.VectorSubcoreMesh(core_axis_name="c", subcore_axis_name="s"),
    scratch_types=[
        pltpu.VMEM((2 * KMAX, CH), jnp.int32),
        pltpu.VMEM((NBUF, CH, HID), jnp.float32),
        pltpu.VMEM_SHARED((AGG_ROWS, HID), jnp.float32),
        pltpu.SemaphoreType.DMA((NBUF,)),
        pltpu.SemaphoreType.DMA((NBUF,)),
    ],
    compiler_params=pltpu.CompilerParams(use_tc_tiling_on_sc=False),
)


def kernel(x, edge_index, edge_type, W1, b1, W2, b2, W3, b3,
           Wl, bl, W0, b0, Wx, bx):
    BR = 1000
    grid = (N // BR,)
    full = lambda shape: pl.BlockSpec(shape, lambda i: (0,) * len(shape))
    bd = (bl + b0 + bx).reshape(1, HID)

    g, d = pl.pallas_call(
        _dense_a_body,
        grid=grid,
        in_specs=[
            pl.BlockSpec((BR, D), lambda i: (i, 0)),
            full((HID, D)), full((1, HID)),
            full((HID, HID)), full((1, HID)),
            full((HID, D)), full((1, D)),
            full((HID, D)), full((HID, D)), full((HID, D)),
            full((1, HID)),
        ],
        out_specs=[
            pl.BlockSpec((BR, HID), lambda i: (i, 0)),
            pl.BlockSpec((BR, HID), lambda i: (i, 0)),
        ],
        out_shape=[
            jax.ShapeDtypeStruct((N, HID), jnp.float32),
            jax.ShapeDtypeStruct((N, HID), jnp.float32),
        ],
    )(x, W1.T, b1.reshape(1, HID), W2, b2.reshape(1, HID),
      W3, b3.reshape(1, D), Wl.T, W0.T, Wx.T, bd)

    partials = _sc_scatter(g, edge_index.reshape(2, 2 * NCHUNK, CH))

    out_t = pl.pallas_call(
        _final_body,
        grid=(1,),
        in_specs=[
            pl.BlockSpec((NC, N, HID), lambda i: (0, 0, 0)),
            pl.BlockSpec((N, HID), lambda i: (0, 0)),
        ],
        out_specs=pl.BlockSpec((HID, N), lambda i: (0, 0)),
        out_shape=jax.ShapeDtypeStruct((HID, N), jnp.float32),
    )(partials, d)
    return out_t.T


# speedup vs baseline: 15.8927x; 1.0518x over previous
import functools

import jax
import jax.numpy as jnp
from jax import lax
from jax.experimental import pallas as pl
from jax.experimental.pallas import tpu as pltpu
from jax.experimental.pallas import tpu_sc as plsc

N = 10000
E = 320000
D = 128
HID = 64

NC = 2
NS = 16
NW = NC * NS
CH = 128
NCHUNK = E // CH
KBASE = NCHUNK // NW
KREM = NCHUNK - NW * KBASE
KMAX = KBASE + 1
AGG_ROWS = 10240
ROWS_PER_TILE = AGG_ROWS // NS
NBUF = 8
PD = 4
NROUND = -(-KMAX // NBUF)


_hp = functools.partial(jnp.dot, preferred_element_type=jnp.float32)
_hpt = functools.partial(lax.dot_general,
                         dimension_numbers=(((1,), (1,)), ((), ())),
                         preferred_element_type=jnp.float32)


def _dense_a1_body(x_ref, w1t, b1, w2, b2, w3, b3, wlt, h_ref, g_ref):
    x = x_ref[...]
    h = jnp.maximum(_hpt(x, w1t[...]) + b1[...], 0.0)
    h = jnp.maximum(_hp(h, w2[...]) + b2[...], 0.0)
    h = _hp(h, w3[...]) + b3[...]
    h_ref[...] = h
    g_ref[...] = _hpt(h, wlt[...])


def _dense_a2_body(x_ref, h_ref, w0t, wxt, bd, d_ref):
    d_ref[...] = (_hpt(h_ref[...], w0t[...]) + _hpt(x_ref[...], wxt[...])
                  + bd[...])


def _final_body(p_ref, d_ref, o_ref):
    o_ref[...] = jnp.maximum(p_ref[0] + p_ref[1] + d_ref[...], 0.0).T


def _sc_scatter_body(g_hbm, idx_hbm, out_hbm,
                     eb, rows_v, agg_sh, gsem, ssem):
    cid = lax.axis_index("c")
    sid = lax.axis_index("s")
    wid = cid * NS + sid
    kw = KBASE + jnp.where(wid >= NW - KREM, 1, 0)
    start = KBASE * wid + jnp.maximum(wid - (NW - KREM), 0)

    zero16 = jnp.zeros((16,), jnp.float32)

    def zbody(i, c):
        for j in range(HID // 16):
            rows_v[0, i, pl.ds(j * 16, 16)] = zero16
        return c

    lax.fori_loop(0, CH, zbody, 0)
    base = sid * ROWS_PER_TILE
    for t in range(ROWS_PER_TILE // CH):
        pltpu.sync_copy(rows_v.at[0], agg_sh.at[pl.ds(base + t * CH, CH)])

    pltpu.sync_copy(idx_hbm.at[0, pl.ds(start, KMAX)], eb.at[pl.ds(0, KMAX)])
    pltpu.sync_copy(idx_hbm.at[1, pl.ds(start, KMAX)],
                    eb.at[pl.ds(KMAX, KMAX)])
    plsc.subcore_barrier()

    def gather(j, bj):
        return pltpu.async_copy(g_hbm.at[eb.at[KMAX + j]], rows_v.at[bj],
                                gsem.at[bj])

    def scatter(i, b):
        return pltpu.async_copy(rows_v.at[b], agg_sh.at[eb.at[i]],
                                ssem.at[b], add=True)

    for b in range(PD):
        gather(b, b)

    def round_body(g, c):
        for b in range(NBUF):
            i = g * NBUF + b
            j = i + PD
            bj = (b + PD) % NBUF

            @pl.when(i < kw)
            def _():
                pltpu.make_async_copy(g_hbm.at[eb.at[KMAX + i]],
                                      rows_v.at[b], gsem.at[b]).wait()
                scatter(i, b)

            @pl.when(jnp.logical_and(j >= NBUF, j < kw))
            def _():
                pltpu.make_async_copy(rows_v.at[bj], agg_sh.at[eb.at[0]],
                                      ssem.at[bj]).wait()

            @pl.when(j < kw)
            def _():
                gather(j, bj)
        return c

    lax.fori_loop(0, NROUND, round_body, 0)
    for b in range(NBUF):
        pltpu.make_async_copy(rows_v.at[b], agg_sh.at[eb.at[0]],
                              ssem.at[b]).wait()
    plsc.subcore_barrier()

    pltpu.sync_copy(agg_sh.at[pl.ds(base, ROWS_PER_TILE)],
                    out_hbm.at[cid, pl.ds(base, ROWS_PER_TILE)])


_sc_scatter = pl.kernel(
    _sc_scatter_body,
    out_type=jax.ShapeDtypeStruct((NC, AGG_ROWS, HID), jnp.float32),
    mesh=plsc.VectorSubcoreMesh(core_axis_name="c", subcore_axis_name="s"),
    scratch_types=[
        pltpu.VMEM((2 * KMAX, CH), jnp.int32),
        pltpu.VMEM((NBUF, CH, HID), jnp.float32),
        pltpu.VMEM_SHARED((AGG_ROWS, HID), jnp.float32),
        pltpu.SemaphoreType.DMA((NBUF,)),
        pltpu.SemaphoreType.DMA((NBUF,)),
    ],
    compiler_params=pltpu.CompilerParams(use_tc_tiling_on_sc=False),
)


def kernel(x, edge_index, edge_type, W1, b1, W2, b2, W3, b3,
           Wl, bl, W0, b0, Wx, bx):
    BR = 1000
    grid = (N // BR,)
    full = lambda shape: pl.BlockSpec(shape, lambda i: (0,) * len(shape))
    bd = (bl + b0 + bx).reshape(1, HID)

    h, g = pl.pallas_call(
        _dense_a1_body,
        grid=grid,
        in_specs=[
            pl.BlockSpec((BR, D), lambda i: (i, 0)),
            full((HID, D)), full((1, HID)),
            full((HID, HID)), full((1, HID)),
            full((HID, D)), full((1, D)),
            full((HID, D)),
        ],
        out_specs=[
            pl.BlockSpec((BR, D), lambda i: (i, 0)),
            pl.BlockSpec((BR, HID), lambda i: (i, 0)),
        ],
        out_shape=[
            jax.ShapeDtypeStruct((N, D), jnp.float32),
            jax.ShapeDtypeStruct((N, HID), jnp.float32),
        ],
    )(x, W1.T, b1.reshape(1, HID), W2, b2.reshape(1, HID),
      W3, b3.reshape(1, D), Wl.T)

    partials = _sc_scatter(g, edge_index[1].reshape(2, NCHUNK, CH))

    d = pl.pallas_call(
        _dense_a2_body,
        grid=grid,
        in_specs=[
            pl.BlockSpec((BR, D), lambda i: (i, 0)),
            pl.BlockSpec((BR, D), lambda i: (i, 0)),
            full((HID, D)), full((HID, D)), full((1, HID)),
        ],
        out_specs=pl.BlockSpec((BR, HID), lambda i: (i, 0)),
        out_shape=jax.ShapeDtypeStruct((N, HID), jnp.float32),
    )(x, h, W0.T, Wx.T, bd)

    out_t = pl.pallas_call(
        _final_body,
        grid=(1,),
        in_specs=[
            pl.BlockSpec((NC, N, HID), lambda i: (0, 0, 0)),
            pl.BlockSpec((N, HID), lambda i: (0, 0)),
        ],
        out_specs=pl.BlockSpec((HID, N), lambda i: (0, 0)),
        out_shape=jax.ShapeDtypeStruct((HID, N), jnp.float32),
    )(partials, d)
    return out_t.T


# speedup vs baseline: 17.2549x; 1.0857x over previous
import functools

import jax
import jax.numpy as jnp
from jax import lax
from jax.experimental import pallas as pl
from jax.experimental.pallas import tpu as pltpu
from jax.experimental.pallas import tpu_sc as plsc

N = 10000
E = 320000
D = 128
HID = 64

NC = 2
NS = 16
NW = NC * NS
CH = 128
NCHUNK = E // CH
KBASE = NCHUNK // NW
KREM = NCHUNK - NW * KBASE
KMAX = KBASE + 1
AGG_ROWS = 10240
ROWS_PER_TILE = AGG_ROWS // NS
NBUF = 8
PD = 4
NROUND = -(-KMAX // NBUF)


_hp = functools.partial(jnp.dot, preferred_element_type=jnp.float32)
_hpt = functools.partial(lax.dot_general,
                         dimension_numbers=(((1,), (1,)), ((), ())),
                         preferred_element_type=jnp.float32)


def _dense_a1_body(x_ref, w1t, b1, w2, b2, w3, b3, wlt, h_ref, g_ref):
    x = x_ref[...]
    h = jnp.maximum(_hpt(x, w1t[...]) + b1[...], 0.0)
    h = jnp.maximum(_hp(h, w2[...]) + b2[...], 0.0)
    h = _hp(h, w3[...]) + b3[...]
    h_ref[...] = h
    g_ref[...] = _hpt(h, wlt[...])


def _dense_a2_body(x_ref, h_ref, w0t, wxt, bd, d_ref):
    d_ref[...] = (_hpt(h_ref[...], w0t[...]) + _hpt(x_ref[...], wxt[...])
                  + bd[...])


def _final_body(p_ref, d_ref, o_ref):
    p = p_ref[0, :, :HID] + p_ref[1, :, :HID]
    o_ref[...] = jnp.maximum(p + d_ref[...], 0.0).T


def _sc_scatter_body(g_hbm, idx_hbm, out_hbm,
                     eb, rows_v, agg_sh, gsem, ssem):
    cid = lax.axis_index("c")
    sid = lax.axis_index("s")
    wid = cid * NS + sid
    kw = KBASE + jnp.where(wid >= NW - KREM, 1, 0)
    start = KBASE * wid + jnp.maximum(wid - (NW - KREM), 0)

    zero16 = jnp.zeros((16,), jnp.float32)

    def zbody(i, c):
        for j in range(HID // 16):
            rows_v[0, i, pl.ds(j * 16, 16)] = zero16
        return c

    lax.fori_loop(0, CH, zbody, 0)
    base = sid * ROWS_PER_TILE
    for t in range(ROWS_PER_TILE // CH):
        pltpu.sync_copy(rows_v.at[0], agg_sh.at[pl.ds(base + t * CH, CH)])

    pltpu.sync_copy(idx_hbm.at[0, pl.ds(start, KMAX)], eb.at[pl.ds(0, KMAX)])
    pltpu.sync_copy(idx_hbm.at[1, pl.ds(start, KMAX)],
                    eb.at[pl.ds(KMAX, KMAX)])
    plsc.subcore_barrier()

    def gather(j, bj):
        return pltpu.async_copy(g_hbm.at[eb.at[KMAX + j]], rows_v.at[bj],
                                gsem.at[bj])

    def scatter(i, b):
        return pltpu.async_copy(rows_v.at[b], agg_sh.at[eb.at[i]],
                                ssem.at[b], add=True)

    for b in range(PD):
        gather(b, b)

    def round_body(g, c):
        for b in range(NBUF):
            i = g * NBUF + b
            j = i + PD
            bj = (b + PD) % NBUF

            @pl.when(i < kw)
            def _():
                pltpu.make_async_copy(g_hbm.at[eb.at[KMAX + i]],
                                      rows_v.at[b], gsem.at[b]).wait()
                scatter(i, b)

            @pl.when(jnp.logical_and(j >= NBUF, j < kw))
            def _():
                pltpu.make_async_copy(rows_v.at[bj], agg_sh.at[eb.at[0]],
                                      ssem.at[bj]).wait()

            @pl.when(j < kw)
            def _():
                gather(j, bj)
        return c

    lax.fori_loop(0, NROUND, round_body, 0)
    for b in range(NBUF):
        pltpu.make_async_copy(rows_v.at[b], agg_sh.at[eb.at[0]],
                              ssem.at[b]).wait()
    plsc.subcore_barrier()

    pltpu.sync_copy(agg_sh.at[pl.ds(base, ROWS_PER_TILE)],
                    out_hbm.at[cid, pl.ds(base, ROWS_PER_TILE),
                               pl.ds(0, HID)])


_sc_scatter = pl.kernel(
    _sc_scatter_body,
    out_type=jax.ShapeDtypeStruct((NC, AGG_ROWS, 2 * HID), jnp.float32),
    mesh=plsc.VectorSubcoreMesh(core_axis_name="c", subcore_axis_name="s"),
    scratch_types=[
        pltpu.VMEM((2 * KMAX, CH), jnp.int32),
        pltpu.VMEM((NBUF, CH, HID), jnp.float32),
        pltpu.VMEM_SHARED((AGG_ROWS, HID), jnp.float32),
        pltpu.SemaphoreType.DMA((NBUF,)),
        pltpu.SemaphoreType.DMA((NBUF,)),
    ],
    compiler_params=pltpu.CompilerParams(use_tc_tiling_on_sc=False),
)


def kernel(x, edge_index, edge_type, W1, b1, W2, b2, W3, b3,
           Wl, bl, W0, b0, Wx, bx):
    BR = 1000
    grid = (N // BR,)
    full = lambda shape: pl.BlockSpec(shape, lambda i: (0,) * len(shape))
    bd = (bl + b0 + bx).reshape(1, HID)

    h, g = pl.pallas_call(
        _dense_a1_body,
        grid=grid,
        in_specs=[
            pl.BlockSpec((BR, D), lambda i: (i, 0)),
            full((HID, D)), full((1, HID)),
            full((HID, HID)), full((1, HID)),
            full((HID, D)), full((1, D)),
            full((HID, D)),
        ],
        out_specs=[
            pl.BlockSpec((BR, D), lambda i: (i, 0)),
            pl.BlockSpec((BR, HID), lambda i: (i, 0)),
        ],
        out_shape=[
            jax.ShapeDtypeStruct((N, D), jnp.float32),
            jax.ShapeDtypeStruct((N, HID), jnp.float32),
        ],
    )(x, W1.T, b1.reshape(1, HID), W2, b2.reshape(1, HID),
      W3, b3.reshape(1, D), Wl.T)

    partials = _sc_scatter(g, edge_index[1].reshape(2, NCHUNK, CH))

    d = pl.pallas_call(
        _dense_a2_body,
        grid=grid,
        in_specs=[
            pl.BlockSpec((BR, D), lambda i: (i, 0)),
            pl.BlockSpec((BR, D), lambda i: (i, 0)),
            full((HID, D)), full((HID, D)), full((1, HID)),
        ],
        out_specs=pl.BlockSpec((BR, HID), lambda i: (i, 0)),
        out_shape=jax.ShapeDtypeStruct((N, HID), jnp.float32),
    )(x, h, W0.T, Wx.T, bd)

    out_t = pl.pallas_call(
        _final_body,
        grid=(1,),
        in_specs=[
            pl.BlockSpec((NC, N, 2 * HID), lambda i: (0, 0, 0)),
            pl.BlockSpec((N, HID), lambda i: (0, 0)),
        ],
        out_specs=pl.BlockSpec((HID, N), lambda i: (0, 0)),
        out_shape=jax.ShapeDtypeStruct((HID, N), jnp.float32),
    )(partials, d)
    return out_t.T


# speedup vs baseline: 18.4581x; 1.0697x over previous
import functools

import jax
import jax.numpy as jnp
from jax import lax
from jax.experimental import pallas as pl
from jax.experimental.pallas import tpu as pltpu
from jax.experimental.pallas import tpu_sc as plsc

N = 10000
E = 320000
D = 128
HID = 64

NC = 2
NS = 16
NW = NC * NS
CH = 128
NCHUNK = E // CH
KBASE = NCHUNK // NW
KREM = NCHUNK - NW * KBASE
KMAX = KBASE + 1
AGG_ROWS = 10240
ROWS_PER_TILE = AGG_ROWS // NS
NBUF = 8
PD = 4
NROUND = -(-KMAX // NBUF)


_hp = functools.partial(jnp.dot, preferred_element_type=jnp.float32)
_hpt = functools.partial(lax.dot_general,
                         dimension_numbers=(((1,), (1,)), ((), ())),
                         preferred_element_type=jnp.float32)


def _dense_a1_body(x_ref, w1t, b1, w2, b2, w3, b3, wlt, g_ref):
    x = x_ref[...]
    h = jnp.maximum(_hpt(x, w1t[...]) + b1[...], 0.0)
    h = jnp.maximum(_hp(h, w2[...]) + b2[...], 0.0)
    h = _hp(h, w3[...]) + b3[...]
    g = _hpt(h, wlt[...])
    g_ref[...] = jnp.concatenate([g, jnp.zeros_like(g)], axis=1)


def _dense_a2_body(x_ref, w1t, b1, w2, b2, w3, b3, w0t, wxt, bd, d_ref):
    x = x_ref[...]
    h = jnp.maximum(_hpt(x, w1t[...]) + b1[...], 0.0)
    h = jnp.maximum(_hp(h, w2[...]) + b2[...], 0.0)
    h = _hp(h, w3[...]) + b3[...]
    d_ref[...] = _hpt(h, w0t[...]) + _hpt(x, wxt[...]) + bd[...]


def _final_body(p_ref, d_ref, o_ref):
    p = p_ref[0, :, :HID] + p_ref[1, :, :HID]
    o_ref[...] = jnp.maximum(p + d_ref[...], 0.0).T


def _sc_scatter_body(g_hbm, idx_hbm, out_hbm,
                     eb, rows_v, agg_sh, gsem, ssem):
    cid = lax.axis_index("c")
    sid = lax.axis_index("s")
    wid = cid * NS + sid
    kw = KBASE + jnp.where(wid >= NW - KREM, 1, 0)
    start = KBASE * wid + jnp.maximum(wid - (NW - KREM), 0)

    zero16 = jnp.zeros((16,), jnp.float32)

    def zbody(i, c):
        for j in range(HID // 16):
            rows_v[0, i, pl.ds(j * 16, 16)] = zero16
        return c

    lax.fori_loop(0, CH, zbody, 0)
    base = sid * ROWS_PER_TILE
    for t in range(ROWS_PER_TILE // CH):
        pltpu.sync_copy(rows_v.at[0], agg_sh.at[pl.ds(base + t * CH, CH)])

    pltpu.sync_copy(idx_hbm.at[0, pl.ds(start, KMAX)], eb.at[pl.ds(0, KMAX)])
    pltpu.sync_copy(idx_hbm.at[1, pl.ds(start, KMAX)],
                    eb.at[pl.ds(KMAX, KMAX)])

    def dbl(r, c):
        for q in range(CH // 16):
            v = eb[KMAX + r, pl.ds(q * 16, 16)]
            eb[KMAX + r, pl.ds(q * 16, 16)] = v + v
        return c

    lax.fori_loop(0, KMAX, dbl, 0)
    plsc.subcore_barrier()

    def gather(j, bj):
        return pltpu.async_copy(g_hbm.at[eb.at[KMAX + j]], rows_v.at[bj],
                                gsem.at[bj])

    def scatter(i, b):
        return pltpu.async_copy(rows_v.at[b], agg_sh.at[eb.at[i]],
                                ssem.at[b], add=True)

    for b in range(PD):
        gather(b, b)

    def round_body(g, c):
        for b in range(NBUF):
            i = g * NBUF + b
            j = i + PD
            bj = (b + PD) % NBUF

            @pl.when(i < kw)
            def _():
                pltpu.make_async_copy(g_hbm.at[eb.at[KMAX + i]],
                                      rows_v.at[b], gsem.at[b]).wait()
                scatter(i, b)

            @pl.when(jnp.logical_and(j >= NBUF, j < kw))
            def _():
                pltpu.make_async_copy(rows_v.at[bj], agg_sh.at[eb.at[0]],
                                      ssem.at[bj]).wait()

            @pl.when(j < kw)
            def _():
                gather(j, bj)
        return c

    lax.fori_loop(0, NROUND, round_body, 0)
    for b in range(NBUF):
        pltpu.make_async_copy(rows_v.at[b], agg_sh.at[eb.at[0]],
                              ssem.at[b]).wait()
    plsc.subcore_barrier()

    pltpu.sync_copy(agg_sh.at[pl.ds(base, ROWS_PER_TILE)],
                    out_hbm.at[cid, pl.ds(base, ROWS_PER_TILE),
                               pl.ds(0, HID)])


_sc_scatter = pl.kernel(
    _sc_scatter_body,
    out_type=jax.ShapeDtypeStruct((NC, AGG_ROWS, 2 * HID), jnp.float32),
    mesh=plsc.VectorSubcoreMesh(core_axis_name="c", subcore_axis_name="s"),
    scratch_types=[
        pltpu.VMEM((2 * KMAX, CH), jnp.int32),
        pltpu.VMEM((NBUF, CH, HID), jnp.float32),
        pltpu.VMEM_SHARED((AGG_ROWS, HID), jnp.float32),
        pltpu.SemaphoreType.DMA((NBUF,)),
        pltpu.SemaphoreType.DMA((NBUF,)),
    ],
    compiler_params=pltpu.CompilerParams(use_tc_tiling_on_sc=False),
)


def kernel(x, edge_index, edge_type, W1, b1, W2, b2, W3, b3,
           Wl, bl, W0, b0, Wx, bx):
    BR = 1000
    grid = (N // BR,)
    full = lambda shape: pl.BlockSpec(shape, lambda i: (0,) * len(shape))
    bd = (bl + b0 + bx).reshape(1, HID)

    g128 = pl.pallas_call(
        _dense_a1_body,
        grid=grid,
        in_specs=[
            pl.BlockSpec((BR, D), lambda i: (i, 0)),
            full((HID, D)), full((1, HID)),
            full((HID, HID)), full((1, HID)),
            full((HID, D)), full((1, D)),
            full((HID, D)),
        ],
        out_specs=pl.BlockSpec((BR, 2 * HID), lambda i: (i, 0)),
        out_shape=jax.ShapeDtypeStruct((N, 2 * HID), jnp.float32),
    )(x, W1.T, b1.reshape(1, HID), W2, b2.reshape(1, HID),
      W3, b3.reshape(1, D), Wl.T)

    partials = _sc_scatter(g128.reshape(2 * N, HID),
                           edge_index[1].reshape(2, NCHUNK, CH))

    d = pl.pallas_call(
        _dense_a2_body,
        grid=grid,
        in_specs=[
            pl.BlockSpec((BR, D), lambda i: (i, 0)),
            full((HID, D)), full((1, HID)),
            full((HID, HID)), full((1, HID)),
            full((HID, D)), full((1, D)),
            full((HID, D)), full((HID, D)), full((1, HID)),
        ],
        out_specs=pl.BlockSpec((BR, HID), lambda i: (i, 0)),
        out_shape=jax.ShapeDtypeStruct((N, HID), jnp.float32),
    )(x, W1.T, b1.reshape(1, HID), W2, b2.reshape(1, HID),
      W3, b3.reshape(1, D), W0.T, Wx.T, bd)

    out_t = pl.pallas_call(
        _final_body,
        grid=(1,),
        in_specs=[
            pl.BlockSpec((NC, N, 2 * HID), lambda i: (0, 0, 0)),
            pl.BlockSpec((N, HID), lambda i: (0, 0)),
        ],
        out_specs=pl.BlockSpec((HID, N), lambda i: (0, 0)),
        out_shape=jax.ShapeDtypeStruct((HID, N), jnp.float32),
    )(partials, d)
    return out_t.T
